# baseline (speedup 1.0000x reference)
import functools

import jax
import jax.numpy as jnp
from jax import lax
from jax.experimental import pallas as pl
from jax.experimental.pallas import tpu as pltpu, tpu_sc as plsc

_NC, _NS = 2, 16
_NW = _NC * _NS
_NBUF = 4
_LANES = 16


def _sc_embed_sum(ex_idx, sk_idx, exercise_table, fused_table, B, L, d):
    bat_per_w = B // _NW
    rows_per_w = bat_per_w * L
    n_groups = bat_per_w // _NBUF
    l_lo = (L // 2 + 7) & ~7
    l_hi = L - l_lo
    mesh = plsc.VectorSubcoreMesh(core_axis_name="c", subcore_axis_name="s")

    @functools.partial(
        pl.kernel,
        out_type=jax.ShapeDtypeStruct((B, L, d), jnp.float32),
        mesh=mesh,
        scratch_types=[
            pltpu.VMEM((rows_per_w,), jnp.int32),
            pltpu.VMEM((rows_per_w,), jnp.int32),
            [pltpu.VMEM((L, d), jnp.float32) for _ in range(_NBUF)],
            [pltpu.SemaphoreType.DMA for _ in range(_NBUF)],
            [pltpu.SemaphoreType.DMA for _ in range(_NBUF)],
            [pltpu.SemaphoreType.DMA for _ in range(_NBUF)],
        ],
        compiler_params=pltpu.CompilerParams(use_tc_tiling_on_sc=False),
    )
    def k(ex_idx_hbm, sk_idx_hbm, ex_tab_hbm, f_tab_hbm, out_hbm,
          eidx_v, fidx_v, bufs, sems_f, sems_e, sems_s):
        wid = lax.axis_index("s") * _NC + lax.axis_index("c")
        base = wid * rows_per_w
        bat0 = wid * bat_per_w
        pltpu.sync_copy(ex_idx_hbm.at[pl.ds(base, rows_per_w)], eidx_v)
        pltpu.sync_copy(sk_idx_hbm.at[pl.ds(base, rows_per_w)], fidx_v)

        lane = jax.lax.iota(jnp.int32, _LANES)

        def fix(i, carry):
            off = i * _LANES
            pos = jax.lax.rem(lane + off, L)
            fidx_v[pl.ds(off, _LANES)] = fidx_v[pl.ds(off, _LANES)] * L + pos
            return carry

        lax.fori_loop(0, rows_per_w // _LANES, fix, 0)

        def gather(tab, idx_ref, j, buf, sem, add):
            pltpu.async_copy(tab.at[idx_ref.at[pl.ds(j * L, l_lo)]],
                             buf.at[pl.ds(0, l_lo)], sem, add=add)
            pltpu.async_copy(tab.at[idx_ref.at[pl.ds(j * L + l_lo, l_hi)]],
                             buf.at[pl.ds(l_lo, l_hi)], sem, add=add)

        def drain(tab, idx_ref, buf, sem):
            pltpu.make_async_copy(tab.at[idx_ref.at[pl.ds(0, l_lo)]],
                                  buf.at[pl.ds(0, l_lo)], sem).wait()
            pltpu.make_async_copy(tab.at[idx_ref.at[pl.ds(0, l_hi)]],
                                  buf.at[pl.ds(l_lo, l_hi)], sem).wait()

        def drain_store(b):
            pltpu.make_async_copy(bufs[b], out_hbm.at[bat0], sems_s[b]).wait()

        def body(g, carry):
            j0 = g * _NBUF
            for b in range(_NBUF):
                @pl.when(g > 0)
                def _():
                    drain_store(b)
                gather(f_tab_hbm, fidx_v, j0 + b, bufs[b], sems_f[b], False)
            for b in range(_NBUF):
                drain(f_tab_hbm, fidx_v, bufs[b], sems_f[b])
                gather(ex_tab_hbm, eidx_v, j0 + b, bufs[b], sems_e[b], True)
            for b in range(_NBUF):
                drain(ex_tab_hbm, eidx_v, bufs[b], sems_e[b])
                pltpu.async_copy(bufs[b], out_hbm.at[bat0 + j0 + b],
                                 sems_s[b])
            return carry

        lax.fori_loop(0, n_groups, body, 0)
        for b in range(_NBUF):
            drain_store(b)

    return k(ex_idx, sk_idx, exercise_table, fused_table)


def kernel(exercises, categories, response, skill, exercise_table,
           position_table, skill_table):
    B, L = exercises.shape
    D = exercise_table.shape[1]

    fused = (skill_table[:, None, :] + position_table[None, :, :]).reshape(-1, D)

    ex_idx = exercises.reshape(-1).astype(jnp.int32)
    sk_idx = skill.reshape(-1).astype(jnp.int32)
    return _sc_embed_sum(ex_idx, sk_idx, exercise_table, fused, B, L, D)


# speedup vs baseline: 1.1244x; 1.1244x over previous
import functools

import jax
import jax.numpy as jnp
from jax import lax
from jax.experimental import pallas as pl
from jax.experimental.pallas import tpu as pltpu, tpu_sc as plsc

_NC, _NS = 2, 16
_NW = _NC * _NS
_NBUF = 4
_LANES = 16
_DP = 128


def _sc_embed_sum(ex_idx, sk_idx, ex_tab, f_tab, B, L, d):
    bat_per_w = B // _NW
    rows_per_w = bat_per_w * L
    l_lo = (L // 2 + 7) & ~7
    l_hi = L - l_lo
    n_chunks = 2 * bat_per_w
    n_groups = n_chunks // _NBUF
    mesh = plsc.VectorSubcoreMesh(core_axis_name="c", subcore_axis_name="s")

    @functools.partial(
        pl.kernel,
        out_type=jax.ShapeDtypeStruct((B * L, _DP), jnp.float32),
        mesh=mesh,
        scratch_types=[
            pltpu.VMEM((rows_per_w,), jnp.int32),
            pltpu.VMEM((rows_per_w,), jnp.int32),
            [pltpu.VMEM((l_lo, _DP), jnp.float32) for _ in range(_NBUF)],
            [pltpu.SemaphoreType.DMA for _ in range(_NBUF)],
            [pltpu.SemaphoreType.DMA for _ in range(_NBUF)],
            [pltpu.SemaphoreType.DMA for _ in range(_NBUF)],
        ],
        compiler_params=pltpu.CompilerParams(use_tc_tiling_on_sc=True),
    )
    def k(ex_idx_hbm, sk_idx_hbm, ex_tab_hbm, f_tab_hbm, out_hbm,
          eidx_v, fidx_v, bufs, sems_f, sems_e, sems_s):
        wid = lax.axis_index("s") * _NC + lax.axis_index("c")
        base = wid * rows_per_w
        bat0 = wid * bat_per_w
        pltpu.sync_copy(ex_idx_hbm.at[pl.ds(base, rows_per_w)], eidx_v)
        pltpu.sync_copy(sk_idx_hbm.at[pl.ds(base, rows_per_w)], fidx_v)

        lane = jax.lax.iota(jnp.int32, _LANES)

        def fix(i, carry):
            off = i * _LANES
            pos = jax.lax.rem(lane + off, L)
            fidx_v[pl.ds(off, _LANES)] = fidx_v[pl.ds(off, _LANES)] * L + pos
            return carry

        lax.fori_loop(0, rows_per_w // _LANES, fix, 0)

        def chunk_geom(c):
            half = jax.lax.rem(c, 2)
            row_off = half * l_lo
            return c // 2, row_off

        def gather(tab, idx_ref, c, buf, sem, add, nr):
            bat, row_off = chunk_geom(c)
            start = bat * L + row_off
            pltpu.async_copy(tab.at[idx_ref.at[pl.ds(start, nr)]],
                             buf.at[pl.ds(0, nr)], sem, add=add)

        def drain(tab, idx_ref, buf, sem, nr):
            pltpu.make_async_copy(tab.at[idx_ref.at[pl.ds(0, nr)]],
                                  buf.at[pl.ds(0, nr)], sem).wait()

        def store(c, buf, sem, nr):
            bat, row_off = chunk_geom(c)
            start = (bat0 + bat) * L + row_off
            pltpu.async_copy(buf.at[pl.ds(0, nr)],
                             out_hbm.at[pl.ds(start, nr)], sem)

        def drain_store(b, nr):
            pltpu.make_async_copy(bufs[b].at[pl.ds(0, nr)],
                                  out_hbm.at[pl.ds(0, nr)],
                                  sems_s[b]).wait()

        def body(g, carry):
            c0 = g * _NBUF
            nrs = [l_lo if b % 2 == 0 else l_hi for b in range(_NBUF)]
            for b in range(_NBUF):
                @pl.when(g > 0)
                def _():
                    drain_store(b, nrs[b])
                gather(f_tab_hbm, fidx_v, c0 + b, bufs[b], sems_f[b], False,
                       nrs[b])
            for b in range(_NBUF):
                drain(f_tab_hbm, fidx_v, bufs[b], sems_f[b], nrs[b])
                gather(ex_tab_hbm, eidx_v, c0 + b, bufs[b], sems_e[b], True,
                       nrs[b])
            for b in range(_NBUF):
                drain(ex_tab_hbm, eidx_v, bufs[b], sems_e[b], nrs[b])
                store(c0 + b, bufs[b], sems_s[b], nrs[b])
            return carry

        lax.fori_loop(0, n_groups, body, 0)
        for b in range(_NBUF):
            drain_store(b, l_lo if b % 2 == 0 else l_hi)

    return k(ex_idx, sk_idx, ex_tab, f_tab)


def kernel(exercises, categories, response, skill, exercise_table,
           position_table, skill_table):
    B, L = exercises.shape
    D = exercise_table.shape[1]

    fused = (skill_table[:, None, :] + position_table[None, :, :]).reshape(-1, D)

    ex_tab = jnp.pad(exercise_table, ((0, 0), (0, _DP - D)))
    f_tab = jnp.pad(fused, ((0, 0), (0, _DP - D)))

    ex_idx = exercises.reshape(-1).astype(jnp.int32)
    sk_idx = skill.reshape(-1).astype(jnp.int32)
    out = _sc_embed_sum(ex_idx, sk_idx, ex_tab, f_tab, B, L, D)
    return out[:, :D].reshape(B, L, D)


# speedup vs baseline: 1.2278x; 1.0920x over previous
import functools

import jax
import jax.numpy as jnp
from jax import lax
from jax.experimental import pallas as pl
from jax.experimental.pallas import tpu as pltpu, tpu_sc as plsc

_NC, _NS = 2, 16
_NW = _NC * _NS
_NBUF = 4
_LANES = 16
_DP = 128


def _sc_embed_sum(ex_idx, sk_idx, ex_tab, f_tab, B, L, d):
    bat_per_w = B // _NW
    rows_per_w = bat_per_w * L
    l_lo = (L // 2 + 7) & ~7
    l_hi = L - l_lo
    n_chunks = 2 * bat_per_w
    n_groups = n_chunks // _NBUF
    mesh = plsc.VectorSubcoreMesh(core_axis_name="c", subcore_axis_name="s")

    @functools.partial(
        pl.kernel,
        out_type=jax.ShapeDtypeStruct((B * L, _DP), jnp.float32),
        mesh=mesh,
        scratch_types=[
            pltpu.VMEM((rows_per_w,), jnp.int32),
            pltpu.VMEM((rows_per_w,), jnp.int32),
            [pltpu.VMEM((l_lo, _DP), jnp.float32) for _ in range(_NBUF)],
            [pltpu.SemaphoreType.DMA for _ in range(_NBUF)],
            [pltpu.SemaphoreType.DMA for _ in range(_NBUF)],
            [pltpu.SemaphoreType.DMA for _ in range(_NBUF)],
        ],
        compiler_params=pltpu.CompilerParams(use_tc_tiling_on_sc=True),
    )
    def k(ex_idx_hbm, sk_idx_hbm, ex_tab_hbm, f_tab_hbm, out_hbm,
          eidx_v, fidx_v, bufs, sems_f, sems_e, sems_s):
        wid = lax.axis_index("s") * _NC + lax.axis_index("c")
        base = wid * rows_per_w
        bat0 = wid * bat_per_w
        pltpu.sync_copy(ex_idx_hbm.at[pl.ds(base, rows_per_w)], eidx_v)
        pltpu.sync_copy(sk_idx_hbm.at[pl.ds(base, rows_per_w)], fidx_v)

        lane = jax.lax.iota(jnp.int32, _LANES)

        def fix(i, carry):
            off = i * _LANES
            pos = jax.lax.rem(lane + off, L)
            fidx_v[pl.ds(off, _LANES)] = fidx_v[pl.ds(off, _LANES)] * L + pos
            return carry

        lax.fori_loop(0, rows_per_w // _LANES, fix, 0)

        def chunk_geom(c):
            half = jax.lax.rem(c, 2)
            row_off = half * l_lo
            return c // 2, row_off

        def gather(tab, idx_ref, c, buf, sem, add, nr):
            bat, row_off = chunk_geom(c)
            start = bat * L + row_off
            pltpu.async_copy(tab.at[idx_ref.at[pl.ds(start, nr)]],
                             buf.at[pl.ds(0, nr)], sem, add=add)

        def drain(tab, idx_ref, buf, sem, nr):
            pltpu.make_async_copy(tab.at[idx_ref.at[pl.ds(0, nr)]],
                                  buf.at[pl.ds(0, nr)], sem).wait()

        def store(c, buf, sem, nr):
            bat, row_off = chunk_geom(c)
            start = (bat0 + bat) * L + row_off
            pltpu.async_copy(buf.at[pl.ds(0, nr)],
                             out_hbm.at[pl.ds(start, nr)], sem)

        def drain_store(b, nr):
            pltpu.make_async_copy(bufs[b].at[pl.ds(0, nr)],
                                  out_hbm.at[pl.ds(0, nr)],
                                  sems_s[b]).wait()

        def body(g, carry):
            c0 = g * _NBUF
            nrs = [l_lo if b % 2 == 0 else l_hi for b in range(_NBUF)]
            for b in range(_NBUF):
                @pl.when(g > 0)
                def _():
                    drain_store(b, nrs[b])
                gather(f_tab_hbm, fidx_v, c0 + b, bufs[b], sems_f[b], False,
                       nrs[b])
            for b in range(_NBUF):
                drain(f_tab_hbm, fidx_v, bufs[b], sems_f[b], nrs[b])
                gather(ex_tab_hbm, eidx_v, c0 + b, bufs[b], sems_e[b], True,
                       nrs[b])
            for b in range(_NBUF):
                drain(ex_tab_hbm, eidx_v, bufs[b], sems_e[b], nrs[b])
                store(c0 + b, bufs[b], sems_s[b], nrs[b])
            return carry

        lax.fori_loop(0, n_groups, body, 0)
        for b in range(_NBUF):
            drain_store(b, l_lo if b % 2 == 0 else l_hi)

    return k(ex_idx, sk_idx, ex_tab, f_tab)


def kernel(exercises, categories, response, skill, exercise_table,
           position_table, skill_table):
    B, L = exercises.shape
    D = exercise_table.shape[1]

    fused = (skill_table[:, None, :] + position_table[None, :, :]).reshape(-1, D)

    pad_mat = jnp.eye(D, _DP, dtype=jnp.float32)
    ex_tab = jax.lax.dot_general(
        exercise_table, pad_mat, (((1,), (0,)), ((), ())),
        precision=jax.lax.Precision.HIGHEST)
    f_tab = jnp.pad(fused, ((0, 0), (0, _DP - D)))

    ex_idx = exercises.reshape(-1).astype(jnp.int32)
    sk_idx = skill.reshape(-1).astype(jnp.int32)
    out = _sc_embed_sum(ex_idx, sk_idx, ex_tab, f_tab, B, L, D)
    return out[:, :D].reshape(B, L, D)


# speedup vs baseline: 1.3993x; 1.1396x over previous
import functools

import jax
import jax.numpy as jnp
from jax import lax
from jax.experimental import pallas as pl
from jax.experimental.pallas import tpu as pltpu, tpu_sc as plsc

_NC, _NS = 2, 16
_NW = _NC * _NS
_NBUF = 4
_LANES = 16
_DP = 128


def _sc_embed_sum(ex_idx, sk_idx, ex_tab, f_tab, B, L, d):
    bat_per_w = B // _NW
    rows_per_w = bat_per_w * L
    l_lo = (L // 2 + 7) & ~7
    l_hi = L - l_lo
    n_chunks = 2 * bat_per_w
    n_groups = n_chunks // _NBUF
    mesh = plsc.VectorSubcoreMesh(core_axis_name="c", subcore_axis_name="s")

    @functools.partial(
        pl.kernel,
        out_type=jax.ShapeDtypeStruct((B * L, _DP), jnp.float32),
        mesh=mesh,
        scratch_types=[
            pltpu.VMEM((rows_per_w,), jnp.int32),
            pltpu.VMEM((rows_per_w,), jnp.int32),
            [pltpu.VMEM((l_lo, _DP), jnp.float32) for _ in range(_NBUF)],
            [pltpu.SemaphoreType.DMA for _ in range(_NBUF)],
            [pltpu.SemaphoreType.DMA for _ in range(_NBUF)],
            [pltpu.SemaphoreType.DMA for _ in range(_NBUF)],
        ],
        compiler_params=pltpu.CompilerParams(use_tc_tiling_on_sc=True),
    )
    def k(ex_idx_hbm, sk_idx_hbm, ex_tab_hbm, f_tab_hbm, out_hbm,
          eidx_v, fidx_v, bufs, sems_f, sems_e, sems_s):
        wid = lax.axis_index("s") * _NC + lax.axis_index("c")
        base = wid * rows_per_w
        bat0 = wid * bat_per_w
        pltpu.sync_copy(ex_idx_hbm.at[pl.ds(base, rows_per_w)], eidx_v)
        pltpu.sync_copy(sk_idx_hbm.at[pl.ds(base, rows_per_w)], fidx_v)

        lane = jax.lax.iota(jnp.int32, _LANES)

        def fix(i, carry):
            off = i * _LANES
            pos = jax.lax.rem(lane + off, L)
            fidx_v[pl.ds(off, _LANES)] = fidx_v[pl.ds(off, _LANES)] * L + pos
            return carry

        lax.fori_loop(0, rows_per_w // _LANES, fix, 0)

        def chunk_geom(c):
            half = jax.lax.rem(c, 2)
            row_off = half * l_lo
            return c // 2, row_off

        def gather(tab, idx_ref, c, buf, sem, add, nr):
            bat, row_off = chunk_geom(c)
            start = bat * L + row_off
            pltpu.async_copy(tab.at[idx_ref.at[pl.ds(start, nr)]],
                             buf.at[pl.ds(0, nr)], sem, add=add)

        def drain(tab, idx_ref, buf, sem, nr):
            pltpu.make_async_copy(tab.at[idx_ref.at[pl.ds(0, nr)]],
                                  buf.at[pl.ds(0, nr)], sem).wait()

        def store(c, buf, sem, nr):
            bat, row_off = chunk_geom(c)
            start = (bat0 + bat) * L + row_off
            pltpu.async_copy(buf.at[pl.ds(0, nr)],
                             out_hbm.at[pl.ds(start, nr)], sem)

        def drain_store(b, nr):
            pltpu.make_async_copy(bufs[b].at[pl.ds(0, nr)],
                                  out_hbm.at[pl.ds(0, nr)],
                                  sems_s[b]).wait()

        def body(g, carry):
            c0 = g * _NBUF
            nrs = [l_lo if b % 2 == 0 else l_hi for b in range(_NBUF)]
            for b in range(_NBUF):
                @pl.when(g > 0)
                def _():
                    drain_store(b, nrs[b])
                gather(f_tab_hbm, fidx_v, c0 + b, bufs[b], sems_f[b], False,
                       nrs[b])
            for b in range(_NBUF):
                drain(f_tab_hbm, fidx_v, bufs[b], sems_f[b], nrs[b])
                gather(ex_tab_hbm, eidx_v, c0 + b, bufs[b], sems_e[b], True,
                       nrs[b])
            for b in range(_NBUF):
                drain(ex_tab_hbm, eidx_v, bufs[b], sems_e[b], nrs[b])
                store(c0 + b, bufs[b], sems_s[b], nrs[b])
            return carry

        lax.fori_loop(0, n_groups, body, 0)
        for b in range(_NBUF):
            drain_store(b, l_lo if b % 2 == 0 else l_hi)

    return k(ex_idx, sk_idx, ex_tab, f_tab)


def kernel(exercises, categories, response, skill, exercise_table,
           position_table, skill_table):
    B, L = exercises.shape
    D = exercise_table.shape[1]

    fused = (skill_table[:, None, :] + position_table[None, :, :]).reshape(-1, D)

    pad_mat = jnp.eye(D, _DP, dtype=jnp.float32)
    ex_tab = jax.lax.dot_general(
        exercise_table, pad_mat, (((1,), (0,)), ((), ())),
        precision=jax.lax.Precision.HIGH)
    f_tab = jnp.pad(fused, ((0, 0), (0, _DP - D)))

    ex_idx = exercises.reshape(-1).astype(jnp.int32)
    sk_idx = skill.reshape(-1).astype(jnp.int32)
    out = _sc_embed_sum(ex_idx, sk_idx, ex_tab, f_tab, B, L, D)
    return out[:, :D].reshape(B, L, D)


# speedup vs baseline: 1.6322x; 1.1665x over previous
import functools

import jax
import jax.numpy as jnp
from jax import lax
from jax.experimental import pallas as pl
from jax.experimental.pallas import tpu as pltpu, tpu_sc as plsc

_NC, _NS = 2, 16
_NW = _NC * _NS
_NBUF = 4
_NSTAGE = 4
_LANES = 16
_DP = 128


def _sc_embed_sum(ex_idx, sk_idx, ex_tab, f_tab, B, L, d):
    bat_per_w = B // _NW
    rows_per_w = bat_per_w * L
    l_lo = (L // 2 + 7) & ~7
    l_hi = L - l_lo
    n_chunks = 2 * bat_per_w
    n_groups = n_chunks // _NBUF
    mesh = plsc.VectorSubcoreMesh(core_axis_name="c", subcore_axis_name="s")

    @functools.partial(
        pl.kernel,
        out_type=jax.ShapeDtypeStruct((B * L, _DP), jnp.float32),
        mesh=mesh,
        scratch_types=[
            pltpu.VMEM((rows_per_w // _NSTAGE,), jnp.int32),
            pltpu.VMEM((rows_per_w // _NSTAGE,), jnp.int32),
            pltpu.VMEM_SHARED((40 * L, _DP), jnp.float32),
            [pltpu.VMEM((l_lo, _DP), jnp.float32) for _ in range(_NBUF)],
            [pltpu.SemaphoreType.DMA for _ in range(_NBUF)],
            [pltpu.SemaphoreType.DMA for _ in range(_NBUF)],
            [pltpu.SemaphoreType.DMA for _ in range(_NBUF)],
        ],
        compiler_params=pltpu.CompilerParams(use_tc_tiling_on_sc=True),
    )
    def k(ex_idx_hbm, sk_idx_hbm, ex_tab_hbm, f_tab_hbm, out_hbm,
          eidx_v, fidx_v, f_shared, bufs, sems_f, sems_e, sems_s):
        wid = lax.axis_index("s") * _NC + lax.axis_index("c")
        base = wid * rows_per_w
        bat0 = wid * bat_per_w
        @pl.when(lax.axis_index("s") == 0)
        def _():
            pltpu.sync_copy(f_tab_hbm, f_shared)

        rows_half = rows_per_w // _NSTAGE
        bat_half = bat_per_w // _NSTAGE
        lane = jax.lax.iota(jnp.int32, _LANES)
        nrs = [l_lo if b % 2 == 0 else l_hi for b in range(_NBUF)]

        def drain(tab, idx_ref, buf, sem, nr):
            pltpu.make_async_copy(tab.at[idx_ref.at[pl.ds(0, nr)]],
                                  buf.at[pl.ds(0, nr)], sem).wait()

        def drain_store(b, nr):
            pltpu.make_async_copy(bufs[b].at[pl.ds(0, nr)],
                                  out_hbm.at[pl.ds(0, nr)],
                                  sems_s[b]).wait()

        for half in range(_NSTAGE):
            pltpu.sync_copy(
                ex_idx_hbm.at[pl.ds(base + half * rows_half, rows_half)],
                eidx_v)
            pltpu.sync_copy(
                sk_idx_hbm.at[pl.ds(base + half * rows_half, rows_half)],
                fidx_v)

            def fix(i, carry):
                off = i * _LANES
                pos = jax.lax.rem(lane + off, L)
                fidx_v[pl.ds(off, _LANES)] = (
                    fidx_v[pl.ds(off, _LANES)] * L + pos)
                return carry

            lax.fori_loop(0, rows_half // _LANES, fix, 0)
            if half == 0:
                plsc.subcore_barrier()

            def gather(tab, idx_ref, cl, buf, sem, add, nr):
                start = (cl // 2) * L + jax.lax.rem(cl, 2) * l_lo
                pltpu.async_copy(tab.at[idx_ref.at[pl.ds(start, nr)]],
                                 buf.at[pl.ds(0, nr)], sem, add=add)

            def store(cl, buf, sem, nr):
                bat = bat0 + half * bat_half + cl // 2
                start = bat * L + jax.lax.rem(cl, 2) * l_lo
                pltpu.async_copy(buf.at[pl.ds(0, nr)],
                                 out_hbm.at[pl.ds(start, nr)], sem)

            def body(g, carry):
                c0 = g * _NBUF
                for b in range(_NBUF):
                    @pl.when(g > 0)
                    def _():
                        drain_store(b, nrs[b])
                    gather(f_shared, fidx_v, c0 + b, bufs[b], sems_f[b],
                           False, nrs[b])
                for b in range(_NBUF):
                    drain(f_shared, fidx_v, bufs[b], sems_f[b], nrs[b])
                    gather(ex_tab_hbm, eidx_v, c0 + b, bufs[b], sems_e[b],
                           True, nrs[b])
                for b in range(_NBUF):
                    drain(ex_tab_hbm, eidx_v, bufs[b], sems_e[b], nrs[b])
                    store(c0 + b, bufs[b], sems_s[b], nrs[b])
                return carry

            lax.fori_loop(0, (2 * bat_half) // _NBUF, body, 0)
            for b in range(_NBUF):
                drain_store(b, nrs[b])

    return k(ex_idx, sk_idx, ex_tab, f_tab)


def kernel(exercises, categories, response, skill, exercise_table,
           position_table, skill_table):
    B, L = exercises.shape
    D = exercise_table.shape[1]

    fused = (skill_table[:, None, :] + position_table[None, :, :]).reshape(-1, D)

    pad_mat = jnp.eye(D, _DP, dtype=jnp.float32)
    ex_tab = jax.lax.dot_general(
        exercise_table, pad_mat, (((1,), (0,)), ((), ())),
        precision=jax.lax.Precision.HIGH)
    f_tab = jnp.pad(fused, ((0, 0), (0, _DP - D)))

    ex_idx = exercises.reshape(-1).astype(jnp.int32)
    sk_idx = skill.reshape(-1).astype(jnp.int32)
    out = _sc_embed_sum(ex_idx, sk_idx, ex_tab, f_tab, B, L, D)
    return out[:, :D].reshape(B, L, D)


# speedup vs baseline: 1.7194x; 1.0534x over previous
import functools

import jax
import jax.numpy as jnp
from jax import lax
from jax.experimental import pallas as pl
from jax.experimental.pallas import tpu as pltpu, tpu_sc as plsc

_NC, _NS = 2, 16
_NW = _NC * _NS
_NBUF = 4
_NSTAGE = 4
_LANES = 16
_DP = 128


def _sc_embed_sum(ex_idx, sk_idx, ex_tab, f_tab, B, L, d):
    bat_per_w = B // _NW
    rows_per_w = bat_per_w * L
    l_lo = (L // 2 + 7) & ~7
    l_hi = L - l_lo
    n_chunks = 2 * bat_per_w
    n_groups = n_chunks // _NBUF
    mesh = plsc.VectorSubcoreMesh(core_axis_name="c", subcore_axis_name="s")

    @functools.partial(
        pl.kernel,
        out_type=jax.ShapeDtypeStruct((B * L, _DP), jnp.float32),
        mesh=mesh,
        scratch_types=[
            pltpu.VMEM((rows_per_w // _NSTAGE,), jnp.int32),
            pltpu.VMEM((rows_per_w // _NSTAGE,), jnp.int32),
            pltpu.VMEM_SHARED((40 * L, _DP), jnp.float32),
            [pltpu.VMEM((l_lo, _DP), jnp.float32) for _ in range(_NBUF)],
            [pltpu.SemaphoreType.DMA for _ in range(_NBUF)],
            [pltpu.SemaphoreType.DMA for _ in range(_NBUF)],
            [pltpu.SemaphoreType.DMA for _ in range(_NBUF)],
        ],
        compiler_params=pltpu.CompilerParams(use_tc_tiling_on_sc=True),
    )
    def k(ex_idx_hbm, sk_idx_hbm, ex_tab_hbm, f_tab_hbm, out_hbm,
          eidx_v, fidx_v, f_shared, bufs, sems_f, sems_e, sems_s):
        wid = lax.axis_index("s") * _NC + lax.axis_index("c")
        base = wid * rows_per_w
        bat0 = wid * bat_per_w
        @pl.when(lax.axis_index("s") == 0)
        def _():
            pltpu.sync_copy(f_tab_hbm, f_shared)

        rows_half = rows_per_w // _NSTAGE
        bat_half = bat_per_w // _NSTAGE
        lane = jax.lax.iota(jnp.int32, _LANES)
        nrs = [l_lo if b % 2 == 0 else l_hi for b in range(_NBUF)]

        def drain(tab, idx_ref, buf, sem, nr):
            pltpu.make_async_copy(tab.at[idx_ref.at[pl.ds(0, nr)]],
                                  buf.at[pl.ds(0, nr)], sem).wait()

        def drain_store(b, nr):
            pltpu.make_async_copy(bufs[b].at[pl.ds(0, nr)],
                                  out_hbm.at[pl.ds(0, nr)],
                                  sems_s[b]).wait()

        for half in range(_NSTAGE):
            pltpu.sync_copy(
                ex_idx_hbm.at[pl.ds(base + half * rows_half, rows_half)],
                eidx_v)
            pltpu.sync_copy(
                sk_idx_hbm.at[pl.ds(base + half * rows_half, rows_half)],
                fidx_v)

            def fix(i, carry):
                off = i * _LANES
                pos = jax.lax.rem(lane + off, L)
                fidx_v[pl.ds(off, _LANES)] = (
                    fidx_v[pl.ds(off, _LANES)] * L + pos)
                return carry

            lax.fori_loop(0, rows_half // _LANES, fix, 0)
            if half == 0:
                plsc.subcore_barrier()

            def gather(tab, idx_ref, cl, buf, sem, add, nr):
                start = (cl // 2) * L + jax.lax.rem(cl, 2) * l_lo
                pltpu.async_copy(tab.at[idx_ref.at[pl.ds(start, nr)]],
                                 buf.at[pl.ds(0, nr)], sem, add=add)

            def store(cl, buf, sem, nr):
                bat = bat0 + half * bat_half + cl // 2
                start = bat * L + jax.lax.rem(cl, 2) * l_lo
                pltpu.async_copy(buf.at[pl.ds(0, nr)],
                                 out_hbm.at[pl.ds(start, nr)], sem)

            def body(g, carry):
                c0 = g * _NBUF
                for b in range(_NBUF):
                    @pl.when(g > 0)
                    def _():
                        drain_store(b, nrs[b])
                    gather(f_shared, fidx_v, c0 + b, bufs[b], sems_f[b],
                           False, nrs[b])
                for b in range(_NBUF):
                    drain(f_shared, fidx_v, bufs[b], sems_f[b], nrs[b])
                    gather(ex_tab_hbm, eidx_v, c0 + b, bufs[b], sems_e[b],
                           True, nrs[b])
                for b in range(_NBUF):
                    drain(ex_tab_hbm, eidx_v, bufs[b], sems_e[b], nrs[b])
                    store(c0 + b, bufs[b], sems_s[b], nrs[b])
                return carry

            lax.fori_loop(0, (2 * bat_half) // _NBUF, body, 0)
            for b in range(_NBUF):
                drain_store(b, nrs[b])

    return k(ex_idx, sk_idx, ex_tab, f_tab)


def kernel(exercises, categories, response, skill, exercise_table,
           position_table, skill_table):
    B, L = exercises.shape
    D = exercise_table.shape[1]

    fused = (skill_table[:, None, :] + position_table[None, :, :]).reshape(-1, D)

    pad_mat = jnp.eye(D, _DP, dtype=jnp.float32)
    ex_tab = jax.lax.dot_general(
        exercise_table, pad_mat, (((1,), (0,)), ((), ())),
        precision=jax.lax.Precision.DEFAULT)
    f_tab = jnp.pad(fused, ((0, 0), (0, _DP - D)))

    ex_idx = exercises.reshape(-1).astype(jnp.int32)
    sk_idx = skill.reshape(-1).astype(jnp.int32)
    out = _sc_embed_sum(ex_idx, sk_idx, ex_tab, f_tab, B, L, D)
    return out[:, :D].reshape(B, L, D)


# speedup vs baseline: 1.7241x; 1.0028x over previous
import functools

import jax
import jax.numpy as jnp
from jax import lax
from jax.experimental import pallas as pl
from jax.experimental.pallas import tpu as pltpu, tpu_sc as plsc

_NC, _NS = 2, 16
_NW = _NC * _NS
_NBUF = 4
_NSTAGE = 4
_LANES = 16
_DP = 128


def _sc_embed_sum(ex_idx, sk_idx, ex_tab, f_tab, B, L, d):
    bat_per_w = B // _NW
    rows_per_w = bat_per_w * L
    l_lo = (L // 2 + 7) & ~7
    l_hi = L - l_lo
    mesh = plsc.VectorSubcoreMesh(core_axis_name="c", subcore_axis_name="s")

    @functools.partial(
        pl.kernel,
        out_type=jax.ShapeDtypeStruct((B * L, _DP), jnp.float32),
        mesh=mesh,
        scratch_types=[
            pltpu.VMEM((rows_per_w // _NSTAGE,), jnp.int32),
            pltpu.VMEM((rows_per_w // _NSTAGE,), jnp.int32),
            pltpu.VMEM_SHARED((40 * L, _DP), jnp.float32),
            [pltpu.VMEM((l_lo, _DP), jnp.float32) for _ in range(_NBUF)],
            [pltpu.SemaphoreType.DMA for _ in range(_NBUF)],
            [pltpu.SemaphoreType.DMA for _ in range(_NBUF)],
            [pltpu.SemaphoreType.DMA for _ in range(_NBUF)],
        ],
        compiler_params=pltpu.CompilerParams(use_tc_tiling_on_sc=True),
    )
    def k(ex_idx_hbm, sk_idx_hbm, ex_tab_hbm, f_tab_hbm, out_hbm,
          eidx_v, fidx_v, f_shared, bufs, sems_f, sems_e, sems_s):
        wid = lax.axis_index("s") * _NC + lax.axis_index("c")
        base = wid * rows_per_w
        bat0 = wid * bat_per_w
        @pl.when(lax.axis_index("s") == 0)
        def _():
            pltpu.sync_copy(f_tab_hbm, f_shared)

        rows_half = rows_per_w // _NSTAGE
        bat_half = bat_per_w // _NSTAGE
        lane = jax.lax.iota(jnp.int32, _LANES)
        nrs = [l_lo if b % 2 == 0 else l_hi for b in range(_NBUF)]

        def drain(tab, idx_ref, buf, sem, nr):
            pltpu.make_async_copy(tab.at[idx_ref.at[pl.ds(0, nr)]],
                                  buf.at[pl.ds(0, nr)], sem).wait()

        def drain_store(b, nr):
            pltpu.make_async_copy(bufs[b].at[pl.ds(0, nr)],
                                  out_hbm.at[pl.ds(0, nr)],
                                  sems_s[b]).wait()

        for half in range(_NSTAGE):
            pltpu.sync_copy(
                ex_idx_hbm.at[pl.ds(base + half * rows_half, rows_half)],
                eidx_v)
            pltpu.sync_copy(
                sk_idx_hbm.at[pl.ds(base + half * rows_half, rows_half)],
                fidx_v)

            def fix(i, carry):
                off = i * _LANES
                pos = jax.lax.rem(lane + off, L)
                fidx_v[pl.ds(off, _LANES)] = (
                    fidx_v[pl.ds(off, _LANES)] * L + pos)
                return carry

            lax.fori_loop(0, rows_half // _LANES, fix, 0)
            if half == 0:
                plsc.subcore_barrier()

            def gather(tab, idx_ref, cl, buf, sem, add, nr):
                start = (cl // 2) * L + jax.lax.rem(cl, 2) * l_lo
                pltpu.async_copy(tab.at[idx_ref.at[pl.ds(start, nr)]],
                                 buf.at[pl.ds(0, nr)], sem, add=add)

            def store(cl, buf, sem, nr):
                bat = bat0 + half * bat_half + cl // 2
                start = bat * L + jax.lax.rem(cl, 2) * l_lo
                pltpu.async_copy(buf.at[pl.ds(0, nr)],
                                 out_hbm.at[pl.ds(start, nr)], sem)

            def body(g, carry):
                c0 = g * _NBUF
                for b in range(_NBUF):
                    @pl.when(g > 0)
                    def _():
                        drain_store(b, nrs[b])
                    gather(f_shared, fidx_v, c0 + b, bufs[b], sems_f[b],
                           False, nrs[b])
                for b in range(_NBUF):
                    drain(f_shared, fidx_v, bufs[b], sems_f[b], nrs[b])
                    gather(ex_tab_hbm, eidx_v, c0 + b, bufs[b], sems_e[b],
                           True, nrs[b])
                for b in range(_NBUF):
                    drain(ex_tab_hbm, eidx_v, bufs[b], sems_e[b], nrs[b])
                    store(c0 + b, bufs[b], sems_s[b], nrs[b])
                return carry

            lax.fori_loop(0, (2 * bat_half) // _NBUF, body, 0)
            for b in range(_NBUF):
                drain_store(b, nrs[b])

    return k(ex_idx, sk_idx, ex_tab, f_tab)


def kernel(exercises, categories, response, skill, exercise_table,
           position_table, skill_table):
    B, L = exercises.shape
    D = exercise_table.shape[1]

    fused = (skill_table[:, None, :] + position_table[None, :, :]).reshape(-1, D)

    pad_mat = jnp.eye(D, _DP, dtype=jnp.float32)
    ex_tab = jax.lax.dot_general(
        exercise_table, pad_mat, (((1,), (0,)), ((), ())),
        precision=jax.lax.Precision.DEFAULT)
    f_tab = jnp.pad(fused, ((0, 0), (0, _DP - D)))

    ex_idx = exercises.reshape(-1).astype(jnp.int32)
    sk_idx = skill.reshape(-1).astype(jnp.int32)
    out = _sc_embed_sum(ex_idx, sk_idx, ex_tab, f_tab, B, L, D)
    return out[:, :D].reshape(B, L, D)
